# baseline (speedup 1.0000x reference)
import functools
import math

import jax
import jax.numpy as jnp
from jax import lax
from jax.experimental import pallas as pl
from jax.experimental.pallas import tpu as pltpu
from jax.experimental.pallas import tpu_sc as plsc

VOCAB = 1000000
D = 64
DP = 128
T = 200
SCALE = math.sqrt(D)
B = 4096

NC = 2
NS = 16
NW = NC * NS
LANES = 16
VPR = D // LANES
CH = 128
ROWS_PER_W = B * T // NW
NCHUNK = ROWS_PER_W // CH
PE2 = T + CH
NIB = 6
NBG = 3
XW = 512


def _pos_encoding():
    position = jnp.arange(0, T, dtype=jnp.float32)[:, None]
    div_term = jnp.exp(
        jnp.arange(0, D, 2, dtype=jnp.float32) * (-(math.log(10000.0) / D)))
    pe = jnp.zeros((T, D), dtype=jnp.float32)
    pe = pe.at[:, 0::2].set(jnp.sin(position * div_term))
    pe = pe.at[:, 1::2].set(jnp.cos(position * div_term))
    return pe


@functools.partial(
    pl.kernel,
    mesh=plsc.VectorSubcoreMesh(core_axis_name="c", subcore_axis_name="s"),
    out_type=jax.ShapeDtypeStruct((B, T, D), jnp.float32),
    scratch_types=[
        pltpu.VMEM((PE2, D), jnp.float32),
        pltpu.VMEM((NIB, CH), jnp.int32),
        pltpu.VMEM((NBG, CH, DP), jnp.float32),
        pltpu.VMEM((2, CH, D), jnp.float32),
        pltpu.SemaphoreType.DMA((NIB,)),
        pltpu.SemaphoreType.DMA((NBG,)),
        pltpu.SemaphoreType.DMA((2,)),
    ],
    compiler_params=pltpu.CompilerParams(use_tc_tiling_on_sc=True),
)
def _emb_kernel(x_hbm, tab_hbm, pe_hbm, out_hbm, pe_v, idx_v, rows_v, out_v,
                sem_i, sem_g, sem_wb):
    wid = lax.axis_index("s") * NC + lax.axis_index("c")
    pltpu.sync_copy(pe_hbm, pe_v)
    out_flat = out_hbm.reshape(B * T, D)
    chunk0 = wid * NCHUNK

    def idx_issue(chunk):
        pltpu.async_copy(
            x_hbm.at[chunk0 + chunk, pl.ds(0, CH)],
            idx_v.at[lax.rem(chunk, NIB)],
            sem_i.at[lax.rem(chunk, NIB)])

    def idx_wait(chunk):
        pltpu.make_async_copy(
            x_hbm.at[chunk0 + chunk, pl.ds(0, CH)],
            idx_v.at[lax.rem(chunk, NIB)],
            sem_i.at[lax.rem(chunk, NIB)]).wait()

    def gather_issue(chunk, buf):
        pltpu.async_copy(
            tab_hbm.at[idx_v.at[lax.rem(chunk, NIB)]],
            rows_v.at[buf], sem_g.at[buf])

    def gather_wait(chunk, buf):
        pltpu.make_async_copy(
            tab_hbm.at[idx_v.at[lax.rem(chunk, NIB)]],
            rows_v.at[buf], sem_g.at[buf]).wait()

    def wb_issue(chunk, buf):
        base = (chunk0 + chunk) * CH
        pltpu.async_copy(out_v.at[buf], out_flat.at[pl.ds(base, CH)],
                         sem_wb.at[buf])

    def wb_wait(chunk, buf):
        base = (chunk0 + chunk) * CH
        pltpu.make_async_copy(
            out_v.at[buf], out_flat.at[pl.ds(base, CH)],
            sem_wb.at[buf]).wait()

    for c in range(NIB - 1):
        idx_issue(c)
    for c in range(NBG - 1):
        idx_wait(c)
        gather_issue(c, c % NBG)

    def chunk_body(i, _):
        b = lax.rem(i, NBG)
        bo = lax.rem(i, 2)
        gather_wait(i, b)

        @pl.when(i >= 2)
        def _():
            wb_wait(i - 2, bo)

        p0 = lax.rem((chunk0 + i) * CH, T)

        def row_body(r8, _):
            r0 = r8 * 8
            for rr in range(8):
                r = r0 + rr
                for d in range(VPR):
                    sl = pl.ds(d * LANES, LANES)
                    out_v[bo, r, sl] = (
                        rows_v[b, r, sl] * SCALE + pe_v[p0 + r, sl])
            return ()

        lax.fori_loop(0, CH // 8, row_body, ())
        wb_issue(i, bo)

        @pl.when(i + NIB - 1 < NCHUNK)
        def _():
            idx_issue(i + NIB - 1)

        @pl.when(i + NBG - 1 < NCHUNK)
        def _():
            idx_wait(i + NBG - 1)
            gather_issue(i + NBG - 1, lax.rem(i + NBG - 1, NBG))

        return ()

    lax.fori_loop(0, NCHUNK, chunk_body, ())
    for j in range(NCHUNK - 2, NCHUNK):
        wb_wait(j, j % 2)


def kernel(x, tok_emb):
    pe = _pos_encoding()
    pe2 = jnp.concatenate([pe, pe[:CH]], axis=0)
    tabp = jnp.pad(tok_emb, ((0, 0), (0, DP - D)))
    x2 = jnp.pad(x.reshape(B * T // CH, CH).astype(jnp.int32),
                 ((0, 0), (0, XW - CH)))
    return _emb_kernel(x2, tabp, pe2)
